# baseline (speedup 1.0000x reference)
import jax
import jax.numpy as jnp
from jax import lax
from jax.experimental import pallas as pl
from jax.experimental.pallas import tpu as pltpu

_N, _C = 16384, 66
_NB = 64
_BLKR = 2048
_SIGMA = 0.25
_THR = 0.001
_SP0, _SP1 = -100.0, -1000.0
_INV_SQRT2 = 0.7071067811865476
_BIG = 3.0e38

_DN = (((1,), (0,)), ((), ()))


def _mm(a, b):
    return lax.dot_general(a, b, _DN,
                           precision=lax.Precision.HIGHEST,
                           preferred_element_type=jnp.float32)


def _tc_body(x_ref, lab_ref, b_ref, out_ref):
    i = pl.program_id(0)
    x = x_ref[...]
    lab = lab_ref[...]
    b = b_ref[...]

    m = jnp.max(x, axis=1, keepdims=True)
    se = jnp.sum(jnp.exp(x - m), axis=1, keepdims=True)
    lse = jnp.log(se) + m

    isp0 = (lab == _SP0).astype(jnp.float32)
    isp1 = (lab == _SP1).astype(jnp.float32)
    pad = isp0 + isp1

    z = (b - lab) * (_INV_SQRT2 / _SIGMA)
    u = lax.erf(z)
    p = 0.5 * (u[1:, :] - u[:-1, :])
    thr = jnp.where(pad > 0.0, _BIG, _THR)
    p = jnp.where(p >= thr, p, 0.0)

    s_mass = jnp.sum(p, axis=0, keepdims=True) + pad

    term_ls = _mm(s_mass, lse)
    pxb = lax.dot_general(p, x[:, 2:], _DN,
                          preferred_element_type=jnp.float32)
    r64 = lax.broadcasted_iota(jnp.int32, (_NB, _NB), 0)
    c64 = lax.broadcasted_iota(jnp.int32, (_NB, _NB), 1)
    d_bins = jnp.sum(jnp.where(r64 == c64, pxb, 0.0))
    isp = jnp.concatenate([isp0, isp1], axis=0)
    sxs = _mm(isp, x[:, :2])
    d_spec = sxs[0, 0] + sxs[1, 1]

    part = (term_ls[0, 0] - d_bins - d_spec) * (1.0 / _N)

    @pl.when(i == 0)
    def _init():
        out_ref[0, 0] = 0.0

    out_ref[0, 0] += part


def kernel(inputs, labels, boundaries):
    grid = _N // _BLKR
    out = pl.pallas_call(
        _tc_body,
        grid=(grid,),
        in_specs=[
            pl.BlockSpec((_BLKR, _C), lambda i: (i, 0)),
            pl.BlockSpec((1, _BLKR), lambda i: (0, i)),
            pl.BlockSpec((_NB + 1, 1), lambda i: (0, 0)),
        ],
        out_specs=pl.BlockSpec(memory_space=pltpu.SMEM),
        out_shape=jax.ShapeDtypeStruct((1, 1), jnp.float32),
        compiler_params=pltpu.CompilerParams(
            dimension_semantics=("arbitrary",)),
    )(inputs, labels.reshape(1, _N), boundaries.reshape(_NB + 1, 1))
    return out[0, 0]


# speedup vs baseline: 1.6083x; 1.6083x over previous
import jax
import jax.numpy as jnp
from jax import lax
from jax.experimental import pallas as pl
from jax.experimental.pallas import tpu as pltpu

_N, _C = 16384, 66
_NB = 64
_BLKL = 8192
_SIGMA = 0.25
_THR = 0.001
_SP0, _SP1 = -100.0, -1000.0
_INV_SQRT2 = 0.7071067811865476
_BIG = 3.0e38


def _tc_body(xb_ref, xs_ref, lab_ref, b_ref, out_ref):
    i = pl.program_id(0)
    xb = xb_ref[...]
    xs = xs_ref[...]
    lab = lab_ref[...]
    b = b_ref[...]

    m = jnp.maximum(jnp.max(xb, axis=0, keepdims=True),
                    jnp.maximum(xs[0:1, :], xs[1:2, :]))
    se = (jnp.sum(jnp.exp(xb - m), axis=0, keepdims=True)
          + jnp.exp(xs[0:1, :] - m) + jnp.exp(xs[1:2, :] - m))
    lse = jnp.log(se) + m

    isp0 = (lab == _SP0).astype(jnp.float32)
    isp1 = (lab == _SP1).astype(jnp.float32)
    pad = isp0 + isp1

    z = (b - lab) * (_INV_SQRT2 / _SIGMA)
    u = lax.erf(z)
    p = 0.5 * (u[1:, :] - u[:-1, :])
    thr = jnp.where(pad > 0.0, _BIG, _THR)
    p = jnp.where(p >= thr, p, 0.0)

    s_mass = jnp.sum(p, axis=0, keepdims=True) + pad
    d_dot = (jnp.sum(p * xb, axis=0, keepdims=True)
             + isp0 * xs[0:1, :] + isp1 * xs[1:2, :])
    part = jnp.sum(lse * s_mass - d_dot) * (1.0 / _N)

    @pl.when(i == 0)
    def _init():
        out_ref[0, 0] = 0.0

    out_ref[0, 0] += part


def kernel(inputs, labels, boundaries):
    xb = inputs[:, 2:].T
    xs = inputs[:, :2].T
    grid = _N // _BLKL
    out = pl.pallas_call(
        _tc_body,
        grid=(grid,),
        in_specs=[
            pl.BlockSpec((_NB, _BLKL), lambda i: (0, i)),
            pl.BlockSpec((2, _BLKL), lambda i: (0, i)),
            pl.BlockSpec((1, _BLKL), lambda i: (0, i)),
            pl.BlockSpec((_NB + 1, 1), lambda i: (0, 0)),
        ],
        out_specs=pl.BlockSpec(memory_space=pltpu.SMEM),
        out_shape=jax.ShapeDtypeStruct((1, 1), jnp.float32),
        compiler_params=pltpu.CompilerParams(
            dimension_semantics=("arbitrary",)),
    )(xb, xs, labels.reshape(1, _N), boundaries.reshape(_NB + 1, 1))
    return out[0, 0]
